# baseline (speedup 1.0000x reference)
import jax
import jax.numpy as jnp
from jax.experimental import pallas as pl
from jax.experimental.pallas import tpu as pltpu

_E = 16
_H = 1024
_FF = 1024
_ALPHA = 1.702
_LIMIT = 7.0
_NTOK = 128


def _moe_kernel(x_ref, rw_ref, rb_ref, bias_ref, gw_ref, uw_ref, dw_ref,
                out_ref, scores_ref, scores_scr, act_scr):
    e = pl.program_id(0)
    p = pl.program_id(1)

    @pl.when((e == 0) & (p == 0))
    def _router():
        x = x_ref[...]
        logits = jax.lax.dot_general(
            x, rw_ref[...], (((1,), (1,)), ((), ())),
            preferred_element_type=jnp.float32) + rb_ref[0][None, :]
        cols = jax.lax.broadcasted_iota(jnp.int32, logits.shape, 1)
        i1 = jnp.argmax(logits, axis=1)
        m1 = jnp.max(logits, axis=1)
        masked = jnp.where(cols == i1[:, None], -jnp.inf, logits)
        i2 = jnp.argmax(masked, axis=1)
        m2 = jnp.max(masked, axis=1)
        t = jnp.exp(m2 - m1)
        p1 = 1.0 / (1.0 + t)
        p2 = t / (1.0 + t)
        scores = (jnp.where(cols == i1[:, None], p1[:, None], 0.0)
                  + jnp.where(cols == i2[:, None], p2[:, None], 0.0))
        scores_scr[...] = scores
        scores_ref[...] = scores

    @pl.when(p == 0)
    def _gate_up():
        x = x_ref[...]
        gb = bias_ref[pl.ds(e, 1), 0:_FF]
        ub = bias_ref[pl.ds(e, 1), _FF:2 * _FF]
        gate = jax.lax.dot_general(
            x, gw_ref[0], (((1,), (1,)), ((), ())),
            preferred_element_type=jnp.float32) + gb
        up = jax.lax.dot_general(
            x, uw_ref[0], (((1,), (1,)), ((), ())),
            preferred_element_type=jnp.float32) + ub
        gate = jnp.minimum(gate, _LIMIT)
        up = jnp.clip(up, -_LIMIT, _LIMIT)
        glu = gate * jax.nn.sigmoid(gate * _ALPHA)
        act_scr[...] = (up + 1.0) * glu

    @pl.when(p == 1)
    def _down():
        cols = jax.lax.broadcasted_iota(jnp.int32, (_NTOK, _E), 1)
        s = jnp.sum(jnp.where(cols == e, scores_scr[...], 0.0), axis=1,
                    keepdims=True)
        db = bias_ref[pl.ds(e, 1), 2 * _FF:2 * _FF + _H]
        y = jax.lax.dot_general(
            act_scr[...], dw_ref[0], (((1,), (1,)), ((), ())),
            preferred_element_type=jnp.float32)
        y = (y + db) * s

        @pl.when(e == 0)
        def _init():
            out_ref[...] = y

        @pl.when(e != 0)
        def _acc():
            out_ref[...] += y


def kernel(hidden_states, router_w, router_b, gate_w, gate_b, up_w, up_b,
           down_w, down_b):
    Bn, Tn, Hn = hidden_states.shape
    x = hidden_states.reshape(-1, Hn)
    rb2 = router_b.reshape(1, _E)
    biases = jnp.concatenate([gate_b, up_b, down_b], axis=1)

    out, scores = pl.pallas_call(
        _moe_kernel,
        grid=(_E, 2),
        in_specs=[
            pl.BlockSpec((_NTOK, _H), lambda e, p: (0, 0)),
            pl.BlockSpec((_E, _H), lambda e, p: (0, 0)),
            pl.BlockSpec((1, _E), lambda e, p: (0, 0)),
            pl.BlockSpec((_E, 2 * _FF + _H), lambda e, p: (0, 0)),
            pl.BlockSpec((1, _FF, _H), lambda e, p: (e, 0, 0)),
            pl.BlockSpec((1, _FF, _H), lambda e, p: (e, 0, 0)),
            pl.BlockSpec((1, _H, _FF), lambda e, p: (e, 0, 0)),
        ],
        out_specs=[
            pl.BlockSpec((_NTOK, _H), lambda e, p: (0, 0)),
            pl.BlockSpec((_NTOK, _E), lambda e, p: (0, 0)),
        ],
        out_shape=[
            jax.ShapeDtypeStruct((_NTOK, _H), jnp.float32),
            jax.ShapeDtypeStruct((_NTOK, _E), jnp.float32),
        ],
        scratch_shapes=[pltpu.VMEM((_NTOK, _E), jnp.float32),
                        pltpu.VMEM((_NTOK, _FF), jnp.float32)],
        compiler_params=pltpu.CompilerParams(
            dimension_semantics=("arbitrary", "arbitrary")),
    )(x, router_w, rb2, biases, gate_w, up_w, down_w)

    return out.reshape(Bn, Tn, Hn), scores


# speedup vs baseline: 1.1351x; 1.1351x over previous
import functools

import jax
import jax.numpy as jnp
from jax import lax
from jax.experimental import pallas as pl
from jax.experimental.pallas import tpu as pltpu
from jax.experimental.pallas import tpu_sc as plsc

_E = 16
_H = 1024
_FF = 1024
_ALPHA = 1.702
_LIMIT = 7.0
_NTOK = 128
_NW = 32
_TPW = _NTOK // _NW


def _router_sc(x_ref, rwt_ref, rb_ref, scores_ref, x_v, rwt_v, rb_v, out_v,
               tmp_v):
    c = lax.axis_index("c")
    s = lax.axis_index("s")
    wid = c * 16 + s
    pltpu.sync_copy(x_ref.at[pl.ds(wid * (_TPW * _H), _TPW * _H)], x_v)
    pltpu.sync_copy(rwt_ref, rwt_v)
    pltpu.sync_copy(rb_ref, rb_v)
    for ti in range(_TPW):
        acc0 = rb_v[...]

        def body(hc, acc, ti=ti):
            xv = x_v[pl.ds(ti * _H + hc * 16, 16)]
            for d in range(16):
                h = hc * 16 + d
                acc = acc + xv[d] * rwt_v[pl.ds(h * _E, _E)]
            return acc

        logits = lax.fori_loop(0, _H // 16, body, acc0)
        lanes_f = lax.iota(jnp.int32, _E).astype(jnp.float32)

        def _rot_reduce(vec, op):
            m = vec
            for k in (1, 2, 4, 8):
                tmp_v[pl.ds(0, _E)] = m
                tmp_v[pl.ds(_E, _E)] = m
                m = op(m, tmp_v[pl.ds(k, _E)])
            return m

        m1 = _rot_reduce(logits, jnp.maximum)
        i1 = _rot_reduce(jnp.where(logits == m1, lanes_f, jnp.float32(99.0)),
                         jnp.minimum)
        masked = jnp.where(lanes_f == i1, jnp.float32(-1e30), logits)
        m2 = _rot_reduce(masked, jnp.maximum)
        i2 = _rot_reduce(jnp.where(masked == m2, lanes_f, jnp.float32(99.0)),
                         jnp.minimum)
        tv = jnp.exp(m2 - m1)
        p1 = 1.0 / (1.0 + tv)
        p2 = tv / (1.0 + tv)
        zero = jnp.zeros((_E,), jnp.float32)
        out_v[pl.ds(ti * _E, _E)] = (jnp.where(lanes_f == i1, p1, zero)
                                     + jnp.where(lanes_f == i2, p2, zero))
    pltpu.sync_copy(out_v, scores_ref.at[pl.ds(wid * (_TPW * _E), _TPW * _E)])


_router = functools.partial(
    pl.kernel,
    out_type=jax.ShapeDtypeStruct((_NTOK * _E,), jnp.float32),
    mesh=plsc.VectorSubcoreMesh(core_axis_name="c", subcore_axis_name="s"),
    scratch_types=[
        pltpu.VMEM((_TPW * _H,), jnp.float32),
        pltpu.VMEM((_H * _E,), jnp.float32),
        pltpu.VMEM((_E,), jnp.float32),
        pltpu.VMEM((_TPW * _E,), jnp.float32),
        pltpu.VMEM((2 * _E,), jnp.float32),
    ],
)(_router_sc)


def _round_bf16(a):
    u = jax.lax.bitcast_convert_type(a, jnp.uint32)
    r = (u + jnp.uint32(0x7FFF) + ((u >> 16) & jnp.uint32(1))) \
        & jnp.uint32(0xFFFF0000)
    return jax.lax.bitcast_convert_type(r, jnp.float32)


def _moe_kernel(x_ref, scores_ref, bias_ref, gw_ref, uw_ref, dw_ref, out_ref):
    e = pl.program_id(0)

    x = x_ref[...]
    cols = jax.lax.broadcasted_iota(jnp.int32, (_NTOK, _E), 1)
    s = jnp.sum(jnp.where(cols == e, scores_ref[...], 0.0), axis=1,
                keepdims=True)

    gb = bias_ref[pl.ds(e, 1), 0:_FF]
    ub = bias_ref[pl.ds(e, 1), _FF:2 * _FF]
    db = bias_ref[pl.ds(e, 1), 2 * _FF:2 * _FF + _H]

    gate = jax.lax.dot_general(
        x, gw_ref[0], (((1,), (1,)), ((), ())),
        preferred_element_type=jnp.float32) + gb
    up = jax.lax.dot_general(
        x, uw_ref[0], (((1,), (1,)), ((), ())),
        preferred_element_type=jnp.float32) + ub
    gate = jnp.minimum(gate, _LIMIT)
    up = jnp.clip(up, -_LIMIT, _LIMIT)
    glu = gate * jax.nn.sigmoid(gate * _ALPHA)
    act = (up + 1.0) * glu
    y = jax.lax.dot_general(
        act, dw_ref[0], (((1,), (1,)), ((), ())),
        preferred_element_type=jnp.float32)
    y = (y + db) * s

    @pl.when(e == 0)
    def _init():
        out_ref[...] = y

    @pl.when(e != 0)
    def _acc():
        out_ref[...] += y


def kernel(hidden_states, router_w, router_b, gate_w, gate_b, up_w, up_b,
           down_w, down_b):
    Bn, Tn, Hn = hidden_states.shape
    x = hidden_states.reshape(-1, Hn)
    biases = jnp.concatenate([gate_b, up_b, down_b], axis=1)

    x_r = _round_bf16(x)
    rwt_r = _round_bf16(router_w).T
    scores_flat = _router(x_r.reshape(_NTOK * _H),
                          rwt_r.reshape(_H * _E), router_b)
    scores = scores_flat.reshape(_NTOK, _E)

    out = pl.pallas_call(
        _moe_kernel,
        grid=(_E,),
        in_specs=[
            pl.BlockSpec((_NTOK, _H), lambda e: (0, 0)),
            pl.BlockSpec((_NTOK, _E), lambda e: (0, 0)),
            pl.BlockSpec((_E, 2 * _FF + _H), lambda e: (0, 0)),
            pl.BlockSpec((1, _FF, _H), lambda e: (e, 0, 0)),
            pl.BlockSpec((1, _FF, _H), lambda e: (e, 0, 0)),
            pl.BlockSpec((1, _H, _FF), lambda e: (e, 0, 0)),
        ],
        out_specs=pl.BlockSpec((_NTOK, _H), lambda e: (0, 0)),
        out_shape=jax.ShapeDtypeStruct((_NTOK, _H), jnp.float32),
        compiler_params=pltpu.CompilerParams(
            dimension_semantics=("arbitrary",)),
    )(x, scores, biases, gate_w, up_w, down_w)

    return out.reshape(Bn, Tn, Hn), scores


# speedup vs baseline: 1.5131x; 1.3330x over previous
import jax
import jax.numpy as jnp
from jax.experimental import pallas as pl
from jax.experimental.pallas import tpu as pltpu

_E = 16
_H = 1024
_FF = 1024
_ALPHA = 1.702
_LIMIT = 7.0
_NTOK = 128


def _moe_kernel(x_ref, rw_ref, rb_ref, bias_ref, gw_ref, uw_ref, dw_ref,
                out_ref, scores_ref, scores_scr):
    e = pl.program_id(0)

    @pl.when(e == 0)
    def _router():
        x = x_ref[...]
        logits = jax.lax.dot_general(
            x, rw_ref[...], (((1,), (1,)), ((), ())),
            preferred_element_type=jnp.float32) + rb_ref[0][None, :]
        cols = jax.lax.broadcasted_iota(jnp.int32, logits.shape, 1)
        i1 = jnp.argmax(logits, axis=1)
        m1 = jnp.max(logits, axis=1)
        masked = jnp.where(cols == i1[:, None], -jnp.inf, logits)
        i2 = jnp.argmax(masked, axis=1)
        m2 = jnp.max(masked, axis=1)
        t = jnp.exp(m2 - m1)
        p1 = 1.0 / (1.0 + t)
        p2 = t / (1.0 + t)
        scores = (jnp.where(cols == i1[:, None], p1[:, None], 0.0)
                  + jnp.where(cols == i2[:, None], p2[:, None], 0.0))
        scores_scr[...] = scores
        scores_ref[...] = scores

    x = x_ref[...]
    cols = jax.lax.broadcasted_iota(jnp.int32, (_NTOK, _E), 1)
    s = jnp.sum(jnp.where(cols == e, scores_scr[...], 0.0), axis=1,
                keepdims=True)

    gb = bias_ref[pl.ds(e, 1), 0:_FF]
    ub = bias_ref[pl.ds(e, 1), _FF:2 * _FF]
    db = bias_ref[pl.ds(e, 1), 2 * _FF:2 * _FF + _H]

    gate = jax.lax.dot_general(
        x, gw_ref[0], (((1,), (1,)), ((), ())),
        preferred_element_type=jnp.float32) + gb
    up = jax.lax.dot_general(
        x, uw_ref[0], (((1,), (1,)), ((), ())),
        preferred_element_type=jnp.float32) + ub
    gate = jnp.minimum(gate, _LIMIT)
    up = jnp.clip(up, -_LIMIT, _LIMIT)
    glu = gate * jax.nn.sigmoid(gate * _ALPHA)
    act = (up + 1.0) * glu
    y = jax.lax.dot_general(
        act, dw_ref[0], (((1,), (1,)), ((), ())),
        preferred_element_type=jnp.float32)
    y = (y + db) * s

    @pl.when(e == 0)
    def _init():
        out_ref[...] = y

    @pl.when(e != 0)
    def _acc():
        out_ref[...] += y


def kernel(hidden_states, router_w, router_b, gate_w, gate_b, up_w, up_b,
           down_w, down_b):
    Bn, Tn, Hn = hidden_states.shape
    x = hidden_states.reshape(-1, Hn)
    rb2 = router_b.reshape(1, _E)
    biases = jnp.concatenate([gate_b, up_b, down_b], axis=1)

    out, scores = pl.pallas_call(
        _moe_kernel,
        grid=(_E,),
        in_specs=[
            pl.BlockSpec((_NTOK, _H), lambda e: (0, 0)),
            pl.BlockSpec((_E, _H), lambda e: (0, 0)),
            pl.BlockSpec((1, _E), lambda e: (0, 0)),
            pl.BlockSpec((_E, 2 * _FF + _H), lambda e: (0, 0)),
            pl.BlockSpec((1, _FF, _H), lambda e: (e, 0, 0)),
            pl.BlockSpec((1, _FF, _H), lambda e: (e, 0, 0)),
            pl.BlockSpec((1, _H, _FF), lambda e: (e, 0, 0)),
        ],
        out_specs=[
            pl.BlockSpec((_NTOK, _H), lambda e: (0, 0)),
            pl.BlockSpec((_NTOK, _E), lambda e: (0, 0)),
        ],
        out_shape=[
            jax.ShapeDtypeStruct((_NTOK, _H), jnp.float32),
            jax.ShapeDtypeStruct((_NTOK, _E), jnp.float32),
        ],
        scratch_shapes=[pltpu.VMEM((_NTOK, _E), jnp.float32)],
        compiler_params=pltpu.CompilerParams(
            dimension_semantics=("arbitrary",)),
    )(x, router_w, rb2, biases, gate_w, up_w, down_w)

    return out.reshape(Bn, Tn, Hn), scores
